# baseline (speedup 1.0000x reference)
import jax
import jax.numpy as jnp
from jax import lax
from jax.experimental import pallas as pl
from jax.experimental.pallas import tpu as pltpu
from jax.experimental.pallas import tpu_sc as plsc

VOCAB = 1000000
DIM = 64

_NC = 2
_NS = 16
_NW = _NC * _NS

_T = 200
_BB = 4096
_B = _T * _BB
_CHUNK = 256
_QPT = _BB // _CHUNK
_BPW = _B // _NW
_NCHUNK = _BPW // _CHUNK


def _gather_body(x_hbm, w_hbm, out_hbm, idx_v, rows, trs, gsems, wsems):
    wid = lax.axis_index("s") * _NC + lax.axis_index("c")
    g0 = wid * _NCHUNK

    pltpu.sync_copy(x_hbm.at[pl.ds(wid * _BPW, _BPW)], idx_v)

    iota16 = jax.lax.broadcasted_iota(jnp.int32, (16,), 0)
    iotas = [iota16 + (16 * jb) for jb in range(_CHUNK // 16)]

    def fire_gather(c, b):
        pltpu.async_copy(
            w_hbm.at[idx_v.at[pl.ds(c * _CHUNK, _CHUNK)]],
            rows[b],
            gsems[b],
        )

    def wait_gather(b):
        pltpu.make_async_copy(out_hbm.at[0, 0, pl.ds(0, _CHUNK)], rows[b],
                              gsems[b]).wait()

    def transpose(b):
        def per_d(d, _):
            dvec = jnp.full((16,), 0, jnp.int32) + d
            for jb in range(_CHUNK // 16):
                vals = plsc.load_gather(rows[b], [iotas[jb], dvec])
                trs[b][d, pl.ds(16 * jb, 16)] = vals
            return ()
        lax.fori_loop(0, DIM, per_d, (), unroll=False)

    def fire_wb(c, b):
        g = g0 + c
        t = lax.shift_right_logical(g, 4)
        col0 = lax.mul(lax.rem(g, _QPT), _CHUNK)
        pltpu.async_copy(trs[b], out_hbm.at[t, :, pl.ds(col0, _CHUNK)],
                         wsems[b])

    def wait_wb(b):
        pltpu.make_async_copy(trs[b], out_hbm.at[0, :, pl.ds(0, _CHUNK)],
                              wsems[b]).wait()

    fire_gather(0, 0)
    fire_gather(1, 1)

    def body(c, _):
        b = lax.rem(c, 2)

        def stage(bb):
            wait_gather(bb)

            @pl.when(c >= 2)
            def _():
                wait_wb(bb)

            transpose(bb)

            @pl.when(c + 2 < _NCHUNK)
            def _():
                fire_gather(c + 2, bb)

            fire_wb(c, bb)

        @pl.when(b == 0)
        def _():
            stage(0)

        @pl.when(b == 1)
        def _():
            stage(1)

        return ()

    lax.fori_loop(0, _NCHUNK, body, (), unroll=False)
    wait_wb(0)
    wait_wb(1)


def _gather(x1, weight):
    return pl.kernel(
        _gather_body,
        out_type=jax.ShapeDtypeStruct((_T, DIM, _BB), jnp.float32),
        mesh=plsc.VectorSubcoreMesh(core_axis_name="c", subcore_axis_name="s"),
        scratch_types=[
            pltpu.VMEM((_BPW,), jnp.int32),
            [pltpu.VMEM((_CHUNK, DIM), jnp.float32) for _ in range(2)],
            [pltpu.VMEM((DIM, _CHUNK), jnp.float32) for _ in range(2)],
            [pltpu.SemaphoreType.DMA for _ in range(2)],
            [pltpu.SemaphoreType.DMA for _ in range(2)],
        ],
        compiler_params=pltpu.CompilerParams(use_tc_tiling_on_sc=False,
                                             needs_layout_passes=False),
    )(x1, weight)


def kernel(x, weight):
    x1 = jnp.swapaxes(x, 0, 1).reshape(_B).astype(jnp.int32)
    out3 = _gather(x1, weight)
    return jnp.transpose(out3, (2, 0, 1))


# speedup vs baseline: 1.3730x; 1.3730x over previous
import jax
import jax.numpy as jnp
from jax import lax
from jax.experimental import pallas as pl
from jax.experimental.pallas import tpu as pltpu
from jax.experimental.pallas import tpu_sc as plsc

VOCAB = 1000000
DIM = 64

_NC = 2
_NS = 16
_NW = _NC * _NS

_T = 200
_BB = 4096
_B = _T * _BB
_CHUNK = 256
_QPT = _BB // _CHUNK
_BPW = _B // _NW
_NCHUNK = _BPW // _CHUNK
_PAD = 65


def _gather_body(x_hbm, w_hbm, out_hbm, idx_v, rows, rowsp, trs, gsems, wsems):
    wid = lax.axis_index("s") * _NC + lax.axis_index("c")
    g0 = wid * _NCHUNK

    pltpu.sync_copy(x_hbm.at[pl.ds(wid * _BPW, _BPW)], idx_v)

    iota16 = jax.lax.broadcasted_iota(jnp.int32, (16,), 0)

    def fire_gather(c, b):
        pltpu.async_copy(
            w_hbm.at[idx_v.at[pl.ds(c * _CHUNK, _CHUNK)]],
            rows[b],
            gsems[b],
        )

    def wait_gather(b):
        pltpu.make_async_copy(out_hbm.at[0, :, pl.ds(0, 2)],
                              trs[b], gsems[b]).wait()

    def pad(b):
        def per_j4(j4, _):
            for jj in range(4):
                j = j4 * 4 + jj
                for q in range(4):
                    rowsp[b][j, pl.ds(q * 16, 16)] = rows[b][j, pl.ds(q * 16, 16)]
            return ()
        lax.fori_loop(0, _CHUNK // 4, per_j4, (), unroll=False)

    def transpose(b):
        def per_d(d, _):
            dvec = iota16 * 0 + d
            r = lax.shift_right_logical(d, 3)
            dr = lax.rem(d, 8)
            for jb in range(_CHUNK // 16):
                vals = plsc.load_gather(rowsp[b].at[pl.ds(jb * 16, 16), :],
                                        [iota16, dvec])
                cc, bm = (jb * 16) // 128, (jb * 16) % 128
                trs[b][r, cc, dr, pl.ds(bm, 16)] = vals
            return ()
        lax.fori_loop(0, DIM, per_d, (), unroll=False)

    def fire_wb(c, b):
        g = g0 + c
        t = lax.shift_right_logical(g, 4)
        c0 = lax.mul(lax.rem(g, _QPT), 2)
        pltpu.async_copy(trs[b], out_hbm.at[t, :, pl.ds(c0, 2)], wsems[b])

    def wait_wb(b):
        pltpu.make_async_copy(trs[b], out_hbm.at[0, :, pl.ds(0, 2)],
                              wsems[b]).wait()

    fire_gather(0, 0)
    fire_gather(1, 1)

    def body(c, _):
        b = lax.rem(c, 2)

        def stage(bb):
            wait_gather(bb)

            @pl.when(c >= 2)
            def _():
                wait_wb(bb)

            pad(bb)
            transpose(bb)

            @pl.when(c + 2 < _NCHUNK)
            def _():
                fire_gather(c + 2, bb)

            fire_wb(c, bb)

        @pl.when(b == 0)
        def _():
            stage(0)

        @pl.when(b == 1)
        def _():
            stage(1)

        return ()

    lax.fori_loop(0, _NCHUNK, body, (), unroll=False)
    wait_wb(0)
    wait_wb(1)


def _gather(x1, weight):
    return pl.kernel(
        _gather_body,
        out_type=jax.ShapeDtypeStruct((_T, 8, _BB // 128, 8, 128), jnp.float32),
        mesh=plsc.VectorSubcoreMesh(core_axis_name="c", subcore_axis_name="s"),
        scratch_types=[
            pltpu.VMEM((_BPW,), jnp.int32),
            [pltpu.VMEM((_CHUNK, DIM), jnp.float32) for _ in range(2)],
            [pltpu.VMEM((_CHUNK, _PAD), jnp.float32) for _ in range(2)],
            [pltpu.VMEM((8, 2, 8, 128), jnp.float32) for _ in range(2)],
            [pltpu.SemaphoreType.DMA for _ in range(2)],
            [pltpu.SemaphoreType.DMA for _ in range(2)],
        ],
        compiler_params=pltpu.CompilerParams(use_tc_tiling_on_sc=False,
                                             needs_layout_passes=False),
    )(x1, weight)


def kernel(x, weight):
    x1 = jnp.swapaxes(x, 0, 1).reshape(_B).astype(jnp.int32)
    out5 = _gather(x1, weight)
    out3 = jnp.transpose(out5, (0, 1, 3, 2, 4)).reshape(_T, DIM, _BB)
    return jnp.transpose(out3, (2, 0, 1))


# speedup vs baseline: 1.6938x; 1.2337x over previous
import jax
import jax.numpy as jnp
from jax import lax
from jax.experimental import pallas as pl
from jax.experimental.pallas import tpu as pltpu
from jax.experimental.pallas import tpu_sc as plsc

VOCAB = 1000000
DIM = 64

_NC = 2
_NS = 16
_NW = _NC * _NS

_T = 200
_BB = 4096
_B = _T * _BB
_CHUNK = 256
_QPT = _BB // _CHUNK
_BPW = _B // _NW
_NCHUNK = _BPW // _CHUNK


def _gather_body(x_hbm, w_hbm, out_hbm, idx_v, rows, gsems, wsems):
    wid = lax.axis_index("s") * _NC + lax.axis_index("c")
    g0 = wid * _NCHUNK

    pltpu.sync_copy(x_hbm.at[pl.ds(wid * _BPW, _BPW)], idx_v)

    def fire_gather(c, b):
        pltpu.async_copy(
            w_hbm.at[idx_v.at[pl.ds(c * _CHUNK, _CHUNK)]],
            rows[b],
            gsems[b],
        )

    def wait_gather(b):
        pltpu.make_async_copy(out_hbm.at[pl.ds(0, _CHUNK)], rows[b],
                              gsems[b]).wait()

    def fire_wb(c, b):
        g = g0 + c
        pltpu.async_copy(rows[b], out_hbm.at[pl.ds(g * _CHUNK, _CHUNK)],
                         wsems[b])

    def wait_wb(b):
        pltpu.make_async_copy(rows[b], out_hbm.at[pl.ds(0, _CHUNK)],
                              wsems[b]).wait()

    fire_gather(0, 0)
    fire_gather(1, 1)

    def body(c, _):
        b = lax.rem(c, 2)

        def stage(bb):
            wait_gather(bb)
            fire_wb(c, bb)
            wait_wb(bb)

            @pl.when(c + 2 < _NCHUNK)
            def _():
                fire_gather(c + 2, bb)

        @pl.when(b == 0)
        def _():
            stage(0)

        @pl.when(b == 1)
        def _():
            stage(1)

        return ()

    lax.fori_loop(0, _NCHUNK, body, (), unroll=False)


def _gather(x1, weight):
    return pl.kernel(
        _gather_body,
        out_type=jax.ShapeDtypeStruct((_B, DIM), jnp.float32),
        mesh=plsc.VectorSubcoreMesh(core_axis_name="c", subcore_axis_name="s"),
        scratch_types=[
            pltpu.VMEM((_BPW,), jnp.int32),
            [pltpu.VMEM((_CHUNK, DIM), jnp.float32) for _ in range(2)],
            [pltpu.SemaphoreType.DMA for _ in range(2)],
            [pltpu.SemaphoreType.DMA for _ in range(2)],
        ],
        compiler_params=pltpu.CompilerParams(use_tc_tiling_on_sc=False,
                                             needs_layout_passes=False),
    )(x1, weight)


def kernel(x, weight):
    x1 = jnp.swapaxes(x, 0, 1).reshape(_B).astype(jnp.int32)
    out_rm = _gather(x1, weight)
    return jnp.transpose(out_rm.reshape(_T, _BB, DIM), (1, 0, 2))
